# baseline (speedup 1.0000x reference)
import functools

import jax
import jax.numpy as jnp
from jax import lax
from jax.experimental import pallas as pl
from jax.experimental.pallas import tpu as pltpu
from jax.experimental.pallas import tpu_sc as plsc


def _sc_relpos_copy(pe, n, off):
    V, D = pe.shape
    info = plsc.get_sparse_core_info()
    NC, NS = info.num_cores, info.num_subcores
    NW = NC * NS
    assert n % NW == 0
    rows_per_w = n // NW

    lo = ((off - n + 1) // 8) * 8
    P = jnp.stack([lax.dynamic_slice_in_dim(pe, lo + d, 2 * n) for d in range(8)])

    mesh = plsc.VectorSubcoreMesh(core_axis_name="c", subcore_axis_name="s")

    @functools.partial(
        pl.kernel,
        out_type=jax.ShapeDtypeStruct((n, n, D), jnp.float32),
        mesh=mesh,
        scratch_types=[pltpu.SemaphoreType.DMA],
    )
    def k(p_hbm, out_hbm, sem):
        wid = lax.axis_index("s") * NC + lax.axis_index("c")
        i0 = wid * rows_per_w
        copies = []
        for r in range(rows_per_w):
            i = i0 + r
            s = off - i
            d = lax.rem(s, 8)
            a = pl.multiple_of(s - d - lo, 8)
            copies.append(
                pltpu.async_copy(
                    p_hbm.at[d, pl.ds(a, n)], out_hbm.at[i], sem
                )
            )
        for c in copies:
            c.wait()

    return k(P)


def kernel(x, q_len, pe):
    n = x.shape[1]
    V = pe.shape[0]
    off = (V + 1) // 2 - 1
    return _sc_relpos_copy(pe, n, off)


# speedup vs baseline: 38.6578x; 38.6578x over previous
import functools

import jax
import jax.numpy as jnp
from jax import lax
from jax.experimental import pallas as pl
from jax.experimental.pallas import tpu as pltpu
from jax.experimental.pallas import tpu_sc as plsc


def _sc_relpos_copy(pe, n, off):
    V, D = pe.shape
    info = plsc.get_sparse_core_info()
    NC, NS = info.num_cores, info.num_subcores
    NW = NC * NS
    assert n % NW == 0
    rows_per_w = n // NW

    C = 64
    NCH = n // C
    K = rows_per_w * NCH

    lo = ((off - n + 1) // 8) * 8
    P = jnp.stack([lax.dynamic_slice_in_dim(pe, lo + d, 2 * n) for d in range(8)])

    mesh = plsc.VectorSubcoreMesh(core_axis_name="c", subcore_axis_name="s")

    @functools.partial(
        pl.kernel,
        out_type=jax.ShapeDtypeStruct((n, n, D), jnp.float32),
        mesh=mesh,
        scratch_types=[
            pltpu.VMEM((C, D), jnp.float32),
            pltpu.VMEM((C, D), jnp.float32),
            pltpu.SemaphoreType.DMA,
            pltpu.SemaphoreType.DMA,
            pltpu.SemaphoreType.DMA,
            pltpu.SemaphoreType.DMA,
        ],
    )
    def k(p_hbm, out_hbm, buf0, buf1, sld0, sld1, sst0, sst1):
        wid = lax.axis_index("s") * NC + lax.axis_index("c")
        i0 = wid * rows_per_w

        def src_dst(step):
            r = lax.div(step, NCH)
            jc = lax.rem(step, NCH) * C
            i = i0 + r
            s = off - i
            d = lax.rem(s, 8)
            a = pl.multiple_of(s - d - lo + jc, 8)
            return p_hbm.at[d, pl.ds(a, C)], out_hbm.at[i, pl.ds(jc, C)]

        def body(g, carry):
            for b, buf, sld, sst in ((0, buf0, sld0, sst0),
                                     (1, buf1, sld1, sst1)):
                step = 2 * g + b
                src, dst = src_dst(step)

                @pl.when(g >= 1)
                def _():
                    pltpu.make_async_copy(src, buf, sst).wait()

                pltpu.async_copy(src, buf, sld).wait()
                pltpu.async_copy(buf, dst, sst)
            return carry

        lax.fori_loop(0, K // 2, body, 0)
        src0, _ = src_dst(0)
        pltpu.make_async_copy(src0, buf0, sst0).wait()
        pltpu.make_async_copy(src0, buf1, sst1).wait()

    return k(P)


def kernel(x, q_len, pe):
    n = x.shape[1]
    V = pe.shape[0]
    off = (V + 1) // 2 - 1
    return _sc_relpos_copy(pe, n, off)
